# baseline (speedup 1.0000x reference)
import functools

import jax
import jax.numpy as jnp
from jax import lax
from jax.experimental import pallas as pl
from jax.experimental.pallas import tpu as pltpu
from jax.experimental.pallas import tpu_sc as plsc

_B = 16384
_D = 64
_H1 = 256
_H2 = 128

_NC = 2
_NS = 16
_NW = _NC * _NS
_ROWS_PER_W = _B // _NW
_CHUNK = 128
_NCHUNK = _ROWS_PER_W // _CHUNK


def _sc_gather(uidx2, iidx2, user_table2, item_table2):
    mesh = plsc.VectorSubcoreMesh(core_axis_name="c", subcore_axis_name="s")

    @functools.partial(
        pl.kernel,
        out_type=(
            jax.ShapeDtypeStruct((_B, 2 * _D), jnp.float32),
            jax.ShapeDtypeStruct((_B, 2 * _D), jnp.float32),
        ),
        mesh=mesh,
        scratch_types=[
            pltpu.VMEM((_NCHUNK, _CHUNK), jnp.int32),
            pltpu.VMEM((_NCHUNK, _CHUNK), jnp.int32),
            pltpu.VMEM((2, _CHUNK, 2 * _D), jnp.float32),
            pltpu.VMEM((2, _CHUNK, 2 * _D), jnp.float32),
            pltpu.SemaphoreType.DMA((2, 2)),
            pltpu.SemaphoreType.DMA((2, 2)),
        ],
    )
    def k(ut_hbm, it_hbm, uidx_hbm, iidx_hbm, uout_hbm, itout_hbm,
          uidx_v, iidx_v, ubuf_v, ibuf_v, gsem, wsem):
        wid = lax.axis_index("s") * _NC + lax.axis_index("c")
        idx_row0 = wid * _NCHUNK
        pltpu.sync_copy(uidx_hbm.at[pl.ds(idx_row0, _NCHUNK)], uidx_v)
        pltpu.sync_copy(iidx_hbm.at[pl.ds(idx_row0, _NCHUNK)], iidx_v)
        base = wid * _ROWS_PER_W

        def gather(j):
            s = j % 2
            return (
                pltpu.async_copy(ut_hbm.at[uidx_v.at[j]], ubuf_v.at[s],
                                 gsem.at[0, s]),
                pltpu.async_copy(it_hbm.at[iidx_v.at[j]], ibuf_v.at[s],
                                 gsem.at[1, s]),
            )

        def writeback(j):
            s = j % 2
            dst = pl.ds(base + j * _CHUNK, _CHUNK)
            return (
                pltpu.async_copy(ubuf_v.at[s], uout_hbm.at[dst],
                                 wsem.at[0, s]),
                pltpu.async_copy(ibuf_v.at[s], itout_hbm.at[dst],
                                 wsem.at[1, s]),
            )

        g = {0: gather(0)}
        wb = {}
        for j in range(_NCHUNK):
            if j + 1 < _NCHUNK:
                if j - 1 >= 0:
                    for c in wb[j - 1]:
                        c.wait()
                g[j + 1] = gather(j + 1)
            for c in g[j]:
                c.wait()
            wb[j] = writeback(j)
        for j in (_NCHUNK - 2, _NCHUNK - 1):
            for c in wb[j]:
                c.wait()

    return k(user_table2, item_table2, uidx2, iidx2)


def _mlp_body(u2_ref, it2_ref, up_ref, ip_ref, w1u_ref, w1i_ref, b1_ref,
              w2_ref, b2_ref, wf_ref, bf_ref, o_ref):
    u2 = u2_ref[...]
    it2 = it2_ref[...]
    u = jnp.where(up_ref[...] > 0, u2[:, _D:], u2[:, :_D])
    it = jnp.where(ip_ref[...] > 0, it2[:, _D:], it2[:, :_D])
    h1 = jnp.dot(u, w1u_ref[...], preferred_element_type=jnp.float32)
    h1 = h1 + jnp.dot(it, w1i_ref[...], preferred_element_type=jnp.float32)
    h1 = jnp.maximum(h1 + b1_ref[...], 0.0)
    h2 = jnp.dot(h1, w2_ref[...], preferred_element_type=jnp.float32)
    h2 = jnp.maximum(h2 + b2_ref[...], 0.0)
    z = jnp.dot(h2, wf_ref[...], preferred_element_type=jnp.float32)
    o_ref[...] = jax.nn.sigmoid(z + bf_ref[...])


def _mlp(u2, it2, uparity, iparity, W1, b1, W2, b2, Wf, bf, blk=2048):
    n_blocks = _B // blk
    return pl.pallas_call(
        _mlp_body,
        grid=(n_blocks,),
        in_specs=[
            pl.BlockSpec((blk, 2 * _D), lambda i: (i, 0)),
            pl.BlockSpec((blk, 2 * _D), lambda i: (i, 0)),
            pl.BlockSpec((blk, 1), lambda i: (i, 0)),
            pl.BlockSpec((blk, 1), lambda i: (i, 0)),
            pl.BlockSpec((_D, _H1), lambda i: (0, 0)),
            pl.BlockSpec((_D, _H1), lambda i: (0, 0)),
            pl.BlockSpec((1, _H1), lambda i: (0, 0)),
            pl.BlockSpec((_H1, _H2), lambda i: (0, 0)),
            pl.BlockSpec((1, _H2), lambda i: (0, 0)),
            pl.BlockSpec((_H2, 1), lambda i: (0, 0)),
            pl.BlockSpec((1, 1), lambda i: (0, 0)),
        ],
        out_specs=pl.BlockSpec((blk, 1), lambda i: (i, 0)),
        out_shape=jax.ShapeDtypeStruct((_B, 1), jnp.float32),
    )(u2, it2, uparity, iparity, W1[:_D], W1[_D:], b1.reshape(1, _H1), W2,
      b2.reshape(1, _H2), Wf, bf.reshape(1, 1))


def kernel(users, items, user_table, item_table, W1, b1, W2, b2, Wf, bf):
    users = users.astype(jnp.int32)
    items = items.astype(jnp.int32)
    uidx2 = (users // 2).reshape(_B // _CHUNK, _CHUNK)
    iidx2 = (items // 2).reshape(_B // _CHUNK, _CHUNK)
    uparity = (users % 2).reshape(_B, 1)
    iparity = (items % 2).reshape(_B, 1)
    ut2 = user_table.reshape(-1, 2 * _D)
    it2 = item_table.reshape(-1, 2 * _D)
    u2, itg2 = _sc_gather(uidx2, iidx2, ut2, it2)
    return _mlp(u2, itg2, uparity, iparity, W1, b1, W2, b2, Wf, bf)
